# baseline (speedup 1.0000x reference)
import jax
import jax.numpy as jnp
from jax.experimental import pallas as pl
from jax.experimental.pallas import tpu as pltpu

N = 10000
IMG_SIZE = 6000
H = 8
DH = 16


def _proj_body(x_ref, w_ref, b_ref, o_ref):
    o_ref[...] = jnp.dot(x_ref[...], w_ref[...],
                         preferred_element_type=jnp.float32) + b_ref[...]


def _proj(x, W, b, bm=1000):
    m, k = x.shape
    return pl.pallas_call(
        _proj_body,
        grid=(m // bm,),
        in_specs=[
            pl.BlockSpec((bm, k), lambda i: (i, 0)),
            pl.BlockSpec((k, 128), lambda i: (0, 0)),
            pl.BlockSpec((128,), lambda i: (0,)),
        ],
        out_specs=pl.BlockSpec((bm, 128), lambda i: (i, 0)),
        out_shape=jax.ShapeDtypeStruct((m, 128), jnp.float32),
    )(x, W, b)


def _gat(h, src, dst, W, al, ar, b):
    Wh = (h @ W).reshape(N, H, DH)
    el = jnp.sum(Wh * al[None], axis=-1)
    er = jnp.sum(Wh * ar[None], axis=-1)
    e = jax.nn.leaky_relu(el[src] + er[dst], negative_slope=0.2)
    m = jax.ops.segment_max(e, dst, num_segments=N)
    m = jnp.where(jnp.isfinite(m), m, 0.0)
    ex = jnp.exp(e - m[dst])
    den = jax.ops.segment_sum(ex, dst, num_segments=N)
    alpha = ex / (den[dst] + 1e-9)
    out = jax.ops.segment_sum(alpha[:, :, None] * Wh[src], dst, num_segments=N)
    out = out + b[None]
    return jax.nn.elu(out).reshape(N, H * DH)


def kernel(img_obj_feat, img_rel_feat, text_obj_feat, text_rel_feat, img_rel, text_tuple, text_word_rel, fc_o_W, fc_o_b, fc_r_W, fc_r_b, fc_w_W, fc_w_b, fc_p_W, fc_p_b, gat0_W, gat0_al, gat0_ar, gat0_b, gat1_W, gat1_al, gat1_ar, gat1_b, sa_W1, sa_b1, sa_W2, pred_W, pred_b, vc_W, vc_b, tc_W, tc_b, ntn_W, ntn_U_W, ntn_U_b, s1_W, s1_b, s2_W, s2_b, s3_W, s3_b):
    h = jnp.concatenate([
        _proj(img_obj_feat, fc_o_W, fc_o_b),
        _proj(img_rel_feat, fc_r_W, fc_r_b),
        _proj(text_obj_feat, fc_w_W, fc_w_b),
        _proj(text_rel_feat, fc_p_W, fc_p_b),
    ], axis=0)
    e2 = jnp.concatenate([text_tuple, text_word_rel], axis=1) + IMG_SIZE
    z0 = _gat(h, img_rel[0], img_rel[1], gat0_W, gat0_al, gat0_ar, gat0_b)
    z1 = _gat(h, e2[0], e2[1], gat1_W, gat1_al, gat1_ar, gat1_b)
    z = jnp.stack([z0, z1], axis=1)
    w = jnp.tanh(z @ sa_W1 + sa_b1) @ sa_W2
    beta = jax.nn.softmax(w.mean(axis=0), axis=0)
    sem = (beta[None] * z).sum(axis=1)
    pred = sem @ pred_W + pred_b
    pred_v = pred[:IMG_SIZE]
    c_v = jnp.tanh(pred_v.mean(axis=0, keepdims=True) @ vc_W + vc_b)
    s_v = jax.nn.sigmoid((pred_v @ c_v.T).squeeze())
    g_v = s_v[None, :] @ pred_v
    pred_t = pred[IMG_SIZE:]
    c_t = jnp.tanh(pred_t.mean(axis=0, keepdims=True) @ tc_W + tc_b)
    s_t = jax.nn.sigmoid((pred_t @ c_t.T).squeeze())
    g_t = s_t[None, :] @ pred_t
    x = jnp.einsum("bi,kij,bj->bk", g_v, ntn_W, g_t) + jnp.concatenate([g_v, g_t], axis=1) @ ntn_U_W + ntn_U_b
    x = jnp.tanh(x)
    x = jax.nn.relu(x @ s1_W + s1_b)
    x = jax.nn.relu(x @ s2_W + s2_b)
    return x @ s3_W + s3_b


# speedup vs baseline: 37.9239x; 37.9239x over previous
import functools

import jax
import jax.numpy as jnp
from jax import lax
from jax.experimental import pallas as pl
from jax.experimental.pallas import tpu as pltpu
from jax.experimental.pallas import tpu_sc as plsc

N = 10000
IMG_SIZE = 6000
H = 8
DH = 16

ROWS = 6016
E_PAD = 161792
EPT = E_PAD // 16
CHUNK = 64
NCHUNK = EPT // CHUNK
RPT = ROWS // 16


def _proj_body(x_ref, w_ref, b_ref, o_ref):
    o_ref[...] = jnp.dot(x_ref[...], w_ref[...],
                         preferred_element_type=jnp.float32) + b_ref[...]


def _proj(x, W, b, bm=1000):
    m, k = x.shape
    return pl.pallas_call(
        _proj_body,
        grid=(m // bm,),
        in_specs=[
            pl.BlockSpec((bm, k), lambda i: (i, 0)),
            pl.BlockSpec((k, 128), lambda i: (0, 0)),
            pl.BlockSpec((128,), lambda i: (0,)),
        ],
        out_specs=pl.BlockSpec((bm, 128), lambda i: (i, 0)),
        out_shape=jax.ShapeDtypeStruct((m, 128), jnp.float32),
    )(x, W, b)


def _zero_phase(idx_s, idx_d, elv, erv, exv, whv, num_sh, den_sh, gsem):
    sub = lax.axis_index("s")
    zv = jnp.zeros((16,), jnp.float32)

    def zwh(i, c):
        whv[i // 8, pl.ds((i % 8) * 16, 16)] = zv
        return c
    lax.fori_loop(0, CHUNK * 8, zwh, 0)

    def zex(i, c):
        exv[i, :] = zv
        return c
    lax.fori_loop(0, CHUNK, zex, 0)

    def zsh(j, c):
        blk = sub + j * 16

        @pl.when(blk < ROWS // CHUNK)
        def _():
            pltpu.sync_copy(whv, num_sh.at[pl.ds(blk * CHUNK, CHUNK)])
            pltpu.sync_copy(exv, den_sh.at[pl.ds(blk * CHUNK, CHUNK)])
        return c
    lax.fori_loop(0, (ROWS // CHUNK + 15) // 16, zsh, 0)


def _edge_phase(src_h, dst_h, el_h, er_h, wh_h,
                idx_s, idx_d, elv, erv, exv, whv, num_sh, den_sh, gsem):
    sub = lax.axis_index("s")

    def chunk(k, c):
        base = sub * EPT + k * CHUNK
        pltpu.sync_copy(src_h.at[pl.ds(base, CHUNK)], idx_s)
        pltpu.sync_copy(dst_h.at[pl.ds(base, CHUNK)], idx_d)
        pltpu.async_copy(el_h.at[idx_s], elv, gsem).wait()
        pltpu.async_copy(er_h.at[idx_d], erv, gsem).wait()
        pltpu.async_copy(wh_h.at[idx_s], whv, gsem).wait()

        def edge(e, cc):
            x = elv[e, pl.ds(0, 16)] + erv[e, pl.ds(0, 16)]
            ex = jnp.exp(jnp.maximum(x, x * 0.2))
            exv[e, :] = ex
            for hh in range(H):
                s = ex[hh]
                whv[e, pl.ds(hh * 16, 16)] = whv[e, pl.ds(hh * 16, 16)] * s
            return cc
        lax.fori_loop(0, CHUNK, edge, 0)

        pltpu.sync_copy(exv, den_sh.at[idx_d], add=True)
        pltpu.sync_copy(whv, num_sh.at[idx_d], add=True)
        return c
    lax.fori_loop(0, NCHUNK, chunk, 0)


def _dump_phase(num_h, den_h, num_sh, den_sh):
    sub = lax.axis_index("s")
    r0 = sub * RPT
    pltpu.sync_copy(num_sh.at[pl.ds(r0, RPT)], num_h.at[pl.ds(r0, RPT)])
    pltpu.sync_copy(den_sh.at[pl.ds(r0, RPT)], den_h.at[pl.ds(r0, RPT)])


@functools.partial(
    pl.kernel,
    out_type=[
        jax.ShapeDtypeStruct((ROWS, 128), jnp.float32),
        jax.ShapeDtypeStruct((ROWS, 16), jnp.float32),
        jax.ShapeDtypeStruct((ROWS, 128), jnp.float32),
        jax.ShapeDtypeStruct((ROWS, 16), jnp.float32),
    ],
    mesh=plsc.VectorSubcoreMesh(core_axis_name="c", subcore_axis_name="s"),
    scratch_types=[
        pltpu.VMEM((CHUNK,), jnp.int32),
        pltpu.VMEM((CHUNK,), jnp.int32),
        pltpu.VMEM((CHUNK, 128), jnp.float32),
        pltpu.VMEM((CHUNK, 128), jnp.float32),
        pltpu.VMEM((CHUNK, 16), jnp.float32),
        pltpu.VMEM((CHUNK, 128), jnp.float32),
        pltpu.VMEM_SHARED((ROWS, 128), jnp.float32),
        pltpu.VMEM_SHARED((ROWS, 16), jnp.float32),
        pltpu.SemaphoreType.DMA,
    ],
)
def _edge_kernel(src0, dst0, el0, er0, wh0, src1, dst1, el1, er1, wh1,
                 num0, den0, num1, den1,
                 idx_s, idx_d, elv, erv, exv, whv, num_sh, den_sh, gsem):
    core = lax.axis_index("c")
    _zero_phase(idx_s, idx_d, elv, erv, exv, whv, num_sh, den_sh, gsem)
    plsc.subcore_barrier()

    @pl.when(core == 0)
    def _():
        _edge_phase(src0, dst0, el0, er0, wh0,
                    idx_s, idx_d, elv, erv, exv, whv, num_sh, den_sh, gsem)

    @pl.when(core == 1)
    def _():
        _edge_phase(src1, dst1, el1, er1, wh1,
                    idx_s, idx_d, elv, erv, exv, whv, num_sh, den_sh, gsem)

    plsc.subcore_barrier()

    @pl.when(core == 0)
    def _():
        _dump_phase(num0, den0, num_sh, den_sh)

    @pl.when(core == 1)
    def _():
        _dump_phase(num1, den1, num_sh, den_sh)


def _pad_edges(src, dst):
    pad = E_PAD - src.shape[0]
    src = jnp.concatenate([src, jnp.zeros((pad,), jnp.int32)])
    dst = jnp.concatenate([dst, jnp.full((pad,), IMG_SIZE, jnp.int32)])
    return src, dst


def _gat_tables(hs, W, al, ar):
    n = hs.shape[0]
    Wh = (hs @ W)
    Whr = Wh.reshape(n, H, DH)
    el = jnp.sum(Whr * al[None], axis=-1)
    er = jnp.sum(Whr * ar[None], axis=-1)
    el16 = jnp.pad(el, ((0, ROWS - n), (0, 128 - H)))
    er16 = jnp.pad(er, ((0, ROWS - n), (0, 128 - H)))
    whp = jnp.pad(Wh, ((0, ROWS - n), (0, 0)))
    return el16, er16, whp


def _gat_out(num, den, b, n):
    denr = jnp.broadcast_to(den[:n, :H, None], (n, H, DH)).reshape(n, 128)
    return jax.nn.elu(num[:n] / (denr + 1e-9) + b.reshape(1, 128))


def kernel(img_obj_feat, img_rel_feat, text_obj_feat, text_rel_feat, img_rel, text_tuple, text_word_rel, fc_o_W, fc_o_b, fc_r_W, fc_r_b, fc_w_W, fc_w_b, fc_p_W, fc_p_b, gat0_W, gat0_al, gat0_ar, gat0_b, gat1_W, gat1_al, gat1_ar, gat1_b, sa_W1, sa_b1, sa_W2, pred_W, pred_b, vc_W, vc_b, tc_W, tc_b, ntn_W, ntn_U_W, ntn_U_b, s1_W, s1_b, s2_W, s2_b, s3_W, s3_b):
    h = jnp.concatenate([
        _proj(img_obj_feat, fc_o_W, fc_o_b),
        _proj(img_rel_feat, fc_r_W, fc_r_b),
        _proj(text_obj_feat, fc_w_W, fc_w_b),
        _proj(text_rel_feat, fc_p_W, fc_p_b),
    ], axis=0)

    el0, er0, wh0 = _gat_tables(h[:IMG_SIZE], gat0_W, gat0_al, gat0_ar)
    el1, er1, wh1 = _gat_tables(h[IMG_SIZE:], gat1_W, gat1_al, gat1_ar)

    src0, dst0 = _pad_edges(img_rel[0], img_rel[1])
    src1, dst1 = _pad_edges(
        jnp.concatenate([text_tuple[0], text_word_rel[0]]),
        jnp.concatenate([text_tuple[1], text_word_rel[1]]))

    num0, den0, num1, den1 = _edge_kernel(
        src0, dst0, el0, er0, wh0, src1, dst1, el1, er1, wh1)

    z0a = _gat_out(num0, den0, gat0_b, IMG_SIZE)
    z1a = _gat_out(num1, den1, gat1_b, N - IMG_SIZE)
    cb0 = jax.nn.elu(gat0_b.reshape(1, 128))
    cb1 = jax.nn.elu(gat1_b.reshape(1, 128))
    z0 = jnp.concatenate([z0a, jnp.broadcast_to(cb0, (N - IMG_SIZE, 128))])
    z1 = jnp.concatenate([jnp.broadcast_to(cb1, (IMG_SIZE, 128)), z1a])

    z = jnp.stack([z0, z1], axis=1)
    w = jnp.tanh(z @ sa_W1 + sa_b1) @ sa_W2
    beta = jax.nn.softmax(w.mean(axis=0), axis=0)
    sem = (beta[None] * z).sum(axis=1)
    pred = sem @ pred_W + pred_b
    pred_v = pred[:IMG_SIZE]
    c_v = jnp.tanh(pred_v.mean(axis=0, keepdims=True) @ vc_W + vc_b)
    s_v = jax.nn.sigmoid((pred_v @ c_v.T).squeeze())
    g_v = s_v[None, :] @ pred_v
    pred_t = pred[IMG_SIZE:]
    c_t = jnp.tanh(pred_t.mean(axis=0, keepdims=True) @ tc_W + tc_b)
    s_t = jax.nn.sigmoid((pred_t @ c_t.T).squeeze())
    g_t = s_t[None, :] @ pred_t
    x = jnp.einsum("bi,kij,bj->bk", g_v, ntn_W, g_t) + jnp.concatenate([g_v, g_t], axis=1) @ ntn_U_W + ntn_U_b
    x = jnp.tanh(x)
    x = jax.nn.relu(x @ s1_W + s1_b)
    x = jax.nn.relu(x @ s2_W + s2_b)
    return x @ s3_W + s3_b
